# Initial kernel scaffold; baseline (speedup 1.0000x reference)
import jax
import jax.numpy as jnp
from jax.experimental import pallas as pl


def kernel(key_table, value_table, depth_table, sibling_table, node_type_table, parent_key_table, kind_table, ln_gamma, ln_beta, token_ids, node_types, depths, sibling_indices, parent_key_ids, kind_ids):
    raise NotImplementedError("write your pallas kernel here")


# speedup vs baseline: 1.0815x; 1.0815x over previous
import dataclasses
import functools

import jax
import jax.numpy as jnp
from jax import lax
from jax.experimental import pallas as pl
from jax.experimental.pallas import tpu as pltpu
from jax.experimental.pallas import tpu_sc as plsc

D = 64
KEY_V = 100000
VAL_V = 1000000
MAX_DEPTH = 64
MAX_SIB = 256
NODE_TYPES = 4
KIND_V = 1000
B = 4096
L = 200
EPS = 1e-5

NL = 16
NW = 32
CHN = 128
N = B * L
CPW = N // (NW * CHN)
NCHUNK = N // CHN
LNB = 4096


def _sc_embed_sum(val_t, key_t, par_t, ds_t, kn_t, pidx):
    mesh = plsc.VectorSubcoreMesh(core_axis_name="c", subcore_axis_name="s")
    cp = pltpu.CompilerParams()
    if "needs_layout_passes" in pltpu.CompilerParams.__dataclass_fields__:
        cp = dataclasses.replace(cp, needs_layout_passes=False)
    if "use_tc_tiling_on_sc" in pltpu.CompilerParams.__dataclass_fields__:
        cp = dataclasses.replace(cp, use_tc_tiling_on_sc=False)

    @functools.partial(
        pl.kernel,
        mesh=mesh,
        compiler_params=cp,
        out_type=jax.ShapeDtypeStruct((N, D), jnp.float32),
        scratch_types=[
            pltpu.VMEM((6, CHN), jnp.int32),
            pltpu.VMEM((CHN, D), jnp.float32),
            pltpu.VMEM((CHN, D), jnp.float32),
            pltpu.VMEM((CHN, D), jnp.float32),
            pltpu.VMEM((CHN, D), jnp.float32),
            pltpu.VMEM((CHN, D), jnp.float32),
            pltpu.SemaphoreType.DMA,
        ],
    )
    def body(val_hbm, key_hbm, par_hbm, ds_hbm, kn_hbm, pidx_hbm, x_hbm,
             ibuf, vbuf, kbuf, pbuf, dbuf, nbuf, sem):
        wid = lax.axis_index("s") * 2 + lax.axis_index("c")

        @pl.loop(0, CPW)
        def _(c):
            cid = wid * CPW + c
            pltpu.sync_copy(pidx_hbm.at[cid], ibuf)
            pairs = [(val_hbm, vbuf), (key_hbm, kbuf), (par_hbm, pbuf),
                     (ds_hbm, dbuf), (kn_hbm, nbuf)]
            cps = [pltpu.async_copy(t.at[ibuf.at[j]], buf, sem)
                   for j, (t, buf) in enumerate(pairs)]
            for cp in cps:
                cp.wait()

            @pl.loop(0, CHN)
            def _(r):
                midx = jnp.full((NL,), r, jnp.int32)
                m = plsc.load_gather(ibuf.at[5], [midx]) != 0
                for cc in range(D // NL):
                    sl = pl.ds(cc * NL, NL)
                    v = vbuf[r, sl]
                    k = kbuf[r, sl]
                    o = jnp.where(m, k, v) + pbuf[r, sl] + dbuf[r, sl] + nbuf[r, sl]
                    vbuf[r, sl] = o

            pltpu.sync_copy(vbuf, x_hbm.at[pl.ds(cid * CHN, CHN)])

    return body(val_t, key_t, par_t, ds_t, kn_t, pidx)


def _ln_body(x_ref, g_ref, b_ref, o_ref):
    x = x_ref[...]
    mu = jnp.mean(x, axis=-1, keepdims=True)
    c = x - mu
    var = jnp.mean(c * c, axis=-1, keepdims=True)
    o_ref[...] = g_ref[...] * (c * lax.rsqrt(var + EPS)) + b_ref[...]


_layernorm = pl.pallas_call(
    _ln_body,
    out_shape=jax.ShapeDtypeStruct((N, D), jnp.float32),
    grid=(N // LNB,),
    in_specs=[
        pl.BlockSpec((LNB, D), lambda i: (i, 0)),
        pl.BlockSpec((1, D), lambda i: (0, 0)),
        pl.BlockSpec((1, D), lambda i: (0, 0)),
    ],
    out_specs=pl.BlockSpec((LNB, D), lambda i: (i, 0)),
)


def kernel(key_table, value_table, depth_table, sibling_table, node_type_table,
           parent_key_table, kind_table, ln_gamma, ln_beta,
           token_ids, node_types, depths, sibling_indices, parent_key_ids,
           kind_ids):
    tok = token_ids.reshape(-1)
    nt = node_types.reshape(-1)
    ival = jnp.clip(tok, 0, VAL_V - 1)
    ikey = jnp.clip(tok, 0, KEY_V - 1)
    ipar = jnp.clip(parent_key_ids.reshape(-1), 0, KEY_V - 1)
    ids = jnp.clip(depths.reshape(-1), 0, MAX_DEPTH - 1) * MAX_SIB + \
        jnp.clip(sibling_indices.reshape(-1), 0, MAX_SIB - 1)
    ikn = jnp.clip(kind_ids.reshape(-1), 0, KIND_V - 1) * NODE_TYPES + \
        jnp.clip(nt, 0, NODE_TYPES - 1)
    mask = ((nt == 0) | (nt == 2)).astype(jnp.int32)

    packed = jnp.stack([ival, ikey, ipar, ids, ikn, mask], axis=0)
    packed = packed.reshape(6, NCHUNK, CHN).transpose(1, 0, 2)

    ds_table = (depth_table[:, None, :] + sibling_table[None, :, :]).reshape(-1, D)
    kn_table = (kind_table[:, None, :] + node_type_table[None, :, :]).reshape(-1, D)

    x = _sc_embed_sum(value_table, key_table, parent_key_table,
                      ds_table, kn_table, packed)
    out = _layernorm(x, ln_gamma.reshape(1, D), ln_beta.reshape(1, D))
    return out.reshape(B, L, D)
